# baseline (speedup 1.0000x reference)
import jax
import jax.numpy as jnp
from jax import lax
from jax.experimental import pallas as pl
from jax.experimental.pallas import tpu as pltpu
from jax.experimental.pallas import tpu_sc as plsc

_LAMB = 0.7
_N = 10000
_D = 128
_E = 320000
_LANES = 16

_NC = 2
_NS = 16
_NW = _NC * _NS
_PER_W = _N // _NW
_EXTRA = (_N - _NW * _PER_W) // 8
_MAXROWS = _PER_W + 8
_CHUNKS = (56, 56, 56, 56, 56, 32, 8)
_OFFS = tuple(sum(_CHUNKS[:i]) for i in range(len(_CHUNKS)))
_NFULL = len(_CHUNKS) - 1
_EPW = (2 * _E) // _NW
_YPW = 400
_YW = _N // _YPW


def _mix_rows(a_v, b_v, lo, hi):
    @plsc.parallel_loop(lo, hi, unroll=4)
    def _(i):
        for j in range(_D // _LANES):
            sl = pl.ds(j * _LANES, _LANES)
            a_v[i, sl] = a_v[i, sl] * _LAMB + b_v[i, sl] * (1.0 - _LAMB)


def _mix_body(x_hbm, pair_hbm, y_hbm, edge_hbm, out_hbm, ynew_hbm, enew_hbm,
              idx_v, a_v, b_v, e_v, y_v, sems, osem, esem, ysem):
    wid = lax.axis_index("s") * _NC + lax.axis_index("c")
    has_extra = wid < _EXTRA
    base = wid * _PER_W + 8 * jnp.minimum(wid, _EXTRA)

    e_load = pltpu.async_copy(edge_hbm.at[pl.ds(wid * _EPW, _EPW)], e_v, esem)

    @pl.when(wid < _YW)
    def _():
        pltpu.async_copy(y_hbm.at[pl.ds(wid * _YPW, _YPW)], y_v, ysem)

    pltpu.sync_copy(pair_hbm.at[pl.ds(base, _PER_W)], idx_v.at[pl.ds(0, _PER_W)])

    @pl.when(has_extra)
    def _():
        pltpu.sync_copy(pair_hbm.at[pl.ds(base + _PER_W, 8)],
                        idx_v.at[pl.ds(_PER_W, 8)])

    def descs(k, make):
        sl = pl.ds(_OFFS[k], _CHUNKS[k])
        return (
            make(x_hbm.at[idx_v.at[sl]], b_v.at[sl], sems[k]),
            make(x_hbm.at[pl.ds(base + _OFFS[k], _CHUNKS[k])], a_v.at[sl],
                 sems[k]),
        )

    def drain(k, copies):
        for c in copies:
            c.wait()
        _mix_rows(a_v, b_v, _OFFS[k], _OFFS[k] + _CHUNKS[k])
        return pltpu.async_copy(
            a_v.at[pl.ds(_OFFS[k], _CHUNKS[k])],
            out_hbm.at[pl.ds(base + _OFFS[k], _CHUNKS[k])], osem)

    copies = [descs(k, pltpu.async_copy) for k in range(_NFULL)]

    @pl.when(has_extra)
    def _():
        descs(_NFULL, pltpu.async_copy)

    stores = [drain(k, copies[k]) for k in range(_NFULL)]

    @pl.when(has_extra)
    def _():
        s = drain(_NFULL, descs(_NFULL, pltpu.make_async_copy))
        s.wait()

    e_load.wait()
    e_store = pltpu.async_copy(e_v, enew_hbm.at[pl.ds(wid * _EPW, _EPW)], osem)

    @pl.when(wid < _YW)
    def _():
        pltpu.make_async_copy(
            y_hbm.at[pl.ds(wid * _YPW, _YPW)], y_v, ysem).wait()
        pltpu.sync_copy(y_v, ynew_hbm.at[pl.ds(wid * _YPW, _YPW)])

    for s in stores:
        s.wait()
    e_store.wait()


@jax.jit
def _node_mixup_sc(x, pair_idx, y, edge_flat):
    mesh = plsc.VectorSubcoreMesh(core_axis_name="c", subcore_axis_name="s")
    call = pl.kernel(
        _mix_body,
        out_type=(
            jax.ShapeDtypeStruct((_N, _D), jnp.float32),
            jax.ShapeDtypeStruct((_N,), jnp.int32),
            jax.ShapeDtypeStruct((2 * _E,), jnp.int32),
        ),
        mesh=mesh,
        scratch_types=[
            pltpu.VMEM((_MAXROWS,), jnp.int32),
            pltpu.VMEM((_MAXROWS, _D), jnp.float32),
            pltpu.VMEM((_MAXROWS, _D), jnp.float32),
            pltpu.VMEM((_EPW,), jnp.int32),
            pltpu.VMEM((_YPW,), jnp.int32),
            [pltpu.SemaphoreType.DMA] * len(_CHUNKS),
            pltpu.SemaphoreType.DMA,
            pltpu.SemaphoreType.DMA,
            pltpu.SemaphoreType.DMA,
        ],
    )
    return call(x, pair_idx, y, edge_flat)


def kernel(x, y, edge_index, pair_idx):
    y32 = y.astype(jnp.int32)
    x_mix, new_y, edge_flat = _node_mixup_sc(
        x, pair_idx, y32, edge_index.reshape(2 * _E))
    return (x_mix, new_y, edge_flat.reshape(2, _E))


# speedup vs baseline: 1.1493x; 1.1493x over previous
import jax
import jax.numpy as jnp
from jax import lax
from jax.experimental import pallas as pl
from jax.experimental.pallas import tpu as pltpu
from jax.experimental.pallas import tpu_sc as plsc

_LAMB = 0.7
_N = 10000
_D = 128
_LANES = 16

_NC = 2
_NS = 16
_NW = _NC * _NS
_PER_W = _N // _NW
_EXTRA = (_N - _NW * _PER_W) // 8
_MAXROWS = _PER_W + 8
_C = 104
_NFULL = _PER_W // _C


def _mix_rows(a_v, b_v, lo, hi):
    @plsc.parallel_loop(lo, hi, unroll=4)
    def _(i):
        for j in range(_D // _LANES):
            sl = pl.ds(j * _LANES, _LANES)
            a_v[i, sl] = a_v[i, sl] * _LAMB + b_v[i, sl] * (1.0 - _LAMB)


def _mix_body(x_hbm, pair_hbm, out_hbm, idx_v, a_v, b_v, sems, esem, lsem,
              osem):
    wid = lax.axis_index("s") * _NC + lax.axis_index("c")
    has_extra = wid < _EXTRA
    base = wid * _PER_W + 8 * jnp.minimum(wid, _EXTRA)

    pltpu.sync_copy(pair_hbm.at[pl.ds(base, _PER_W)],
                    idx_v.at[pl.ds(0, _PER_W)])

    @pl.when(has_extra)
    def _():
        pltpu.sync_copy(pair_hbm.at[pl.ds(base + _PER_W, 8)],
                        idx_v.at[pl.ds(_PER_W, 8)])

    gathers = [
        pltpu.async_copy(x_hbm.at[idx_v.at[pl.ds(k * _C, _C)]],
                         b_v.at[pl.ds(k * _C, _C)], sems[k])
        for k in range(_NFULL)
    ]

    def extra_descs(make):
        sl = pl.ds(_PER_W, 8)
        return (
            make(x_hbm.at[idx_v.at[sl]], b_v.at[sl], esem),
            make(x_hbm.at[pl.ds(base + _PER_W, 8)], a_v.at[sl], esem),
        )

    @pl.when(has_extra)
    def _():
        extra_descs(pltpu.async_copy)

    pltpu.async_copy(x_hbm.at[pl.ds(base, _PER_W)],
                     a_v.at[pl.ds(0, _PER_W)], lsem).wait()

    stores = []
    for k in range(_NFULL):
        gathers[k].wait()
        _mix_rows(a_v, b_v, k * _C, (k + 1) * _C)
        stores.append(pltpu.async_copy(
            a_v.at[pl.ds(k * _C, _C)],
            out_hbm.at[pl.ds(base + k * _C, _C)], osem))

    @pl.when(has_extra)
    def _():
        for c in extra_descs(pltpu.make_async_copy):
            c.wait()
        _mix_rows(a_v, b_v, _PER_W, _MAXROWS)
        pltpu.sync_copy(a_v.at[pl.ds(_PER_W, 8)],
                        out_hbm.at[pl.ds(base + _PER_W, 8)])

    for s in stores:
        s.wait()


@jax.jit
def _node_mixup_sc(x, pair_idx):
    mesh = plsc.VectorSubcoreMesh(core_axis_name="c", subcore_axis_name="s")
    call = pl.kernel(
        _mix_body,
        out_type=jax.ShapeDtypeStruct((_N, _D), jnp.float32),
        mesh=mesh,
        scratch_types=[
            pltpu.VMEM((_MAXROWS,), jnp.int32),
            pltpu.VMEM((_MAXROWS, _D), jnp.float32),
            pltpu.VMEM((_MAXROWS, _D), jnp.float32),
            [pltpu.SemaphoreType.DMA] * _NFULL,
            pltpu.SemaphoreType.DMA,
            pltpu.SemaphoreType.DMA,
            pltpu.SemaphoreType.DMA,
        ],
    )
    return call(x, pair_idx)


def kernel(x, y, edge_index, pair_idx):
    x_mix = _node_mixup_sc(x, pair_idx)
    new_y = y.astype(jnp.int32)
    return (x_mix, new_y, edge_index)
